# baseline (speedup 1.0000x reference)
import functools

import jax
import jax.numpy as jnp
from jax import lax
from jax.experimental import pallas as pl
from jax.experimental.pallas import tpu as pltpu
from jax.experimental.pallas import tpu_sc as plsc

N = 10000
NP = 10240
E = 320000
D = 128
HD = 64
NC = 2
NS = 16
NW = NC * NS
CH = 128
K = 8
EP = NW * 10240
EPW = EP // NS
ITERS = EPW // CH
RPT = NP // NS
RZ = 128
DEGW = 16
CHD = 128
EPWD = EP // NW
ITD = EPWD // CHD

_MESH = dict(core_axis_name="c", subcore_axis_name="s")


def _zero_acc(zer_h, buf_v, acc_sh, s, rz):
    pltpu.sync_copy(zer_h, buf_v)
    for k in range(RPT // rz):
        pltpu.sync_copy(buf_v, acc_sh.at[pl.ds(s * RPT + k * rz, rz)])


def _read_out(out_h, buf_v, acc_sh, c, s, rz):
    for k in range(RPT // rz):
        b = s * RPT + k * rz
        pltpu.sync_copy(acc_sh.at[pl.ds(b, rz)], buf_v)
        pltpu.sync_copy(buf_v, out_h.at[c, pl.ds(b, rz)])


@functools.partial(
    pl.kernel,
    out_type=jax.ShapeDtypeStruct((NC, NP, DEGW), jnp.float32),
    mesh=plsc.VectorSubcoreMesh(**_MESH),
    scratch_types=[
        pltpu.VMEM((ITD, CHD), jnp.int32),
        pltpu.VMEM((CHD, DEGW), jnp.float32),
        pltpu.VMEM((RZ, DEGW), jnp.float32),
        pltpu.VMEM_SHARED((NP, DEGW), jnp.float32),
        pltpu.SemaphoreType.DMA,
    ],
    compiler_params=pltpu.CompilerParams(use_tc_tiling_on_sc=False),
    name="sc_degree",
)
def _sc_degree(dst_h, ones_h, zer_h, out_h, dstb_v, ones_v, buf_v, acc_sh,
               ssem):
    c = lax.axis_index("c")
    s = lax.axis_index("s")
    wid = s * NC + c
    pltpu.sync_copy(ones_h, ones_v)
    pltpu.sync_copy(dst_h.at[wid], dstb_v)
    _zero_acc(zer_h, buf_v, acc_sh, s, RZ)
    plsc.subcore_barrier()

    def step(g, carry):
        descs = [
            pltpu.async_copy(ones_v, acc_sh.at[dstb_v.at[g * K + b]], ssem,
                             add=True)
            for b in range(K)
        ]
        for d in descs:
            d.wait()
        return carry

    lax.fori_loop(0, ITD // K, step, 0)
    plsc.subcore_barrier()
    _read_out(out_h, buf_v, acc_sh, c, s, RZ)


@functools.partial(
    pl.kernel,
    out_type=jax.ShapeDtypeStruct((NC, NP, HD), jnp.float32),
    mesh=plsc.VectorSubcoreMesh(**_MESH),
    scratch_types=[
        pltpu.VMEM((EPW,), jnp.int32),
        pltpu.VMEM((ITERS, CH), jnp.int32),
        [pltpu.VMEM((CH, HD), jnp.float32)] * 2,
        pltpu.VMEM_SHARED((NP, HD), jnp.float32),
        pltpu.SemaphoreType.DMA,
        pltpu.SemaphoreType.DMA,
        pltpu.SemaphoreType.DMA,
        pltpu.SemaphoreType.DMA,
    ],
    compiler_params=pltpu.CompilerParams(use_tc_tiling_on_sc=False),
    name="sc_agg",
)
def _sc_agg(tab_h, src_h, dst_h, zer_h, out_h,
            srcb_v, dstb_v, rows, acc_sh, gsemA, gsemB, ssemA, ssemB):
    c = lax.axis_index("c")
    s = lax.axis_index("s")
    pltpu.sync_copy(src_h.at[c, s], srcb_v)
    pltpu.sync_copy(dst_h.at[s], dstb_v)
    _zero_acc(zer_h, rows[0], acc_sh, s, RZ)
    plsc.subcore_barrier()

    def step(i, carry):
        k0 = 2 * i
        k1 = 2 * i + 1
        gA = pltpu.async_copy(
            tab_h.at[srcb_v.at[pl.ds(k0 * CH, CH)]], rows[0], gsemA)
        gB = pltpu.async_copy(
            tab_h.at[srcb_v.at[pl.ds(k1 * CH, CH)]], rows[1], gsemB)
        gA.wait()
        sA = pltpu.async_copy(rows[0], acc_sh.at[dstb_v.at[k0]], ssemA,
                              add=True)
        gB.wait()
        sB = pltpu.async_copy(rows[1], acc_sh.at[dstb_v.at[k1]], ssemB,
                              add=True)
        sA.wait()
        sB.wait()
        return carry

    lax.fori_loop(0, ITERS // 2, step, 0)
    plsc.subcore_barrier()
    _read_out(out_h, rows[0], acc_sh, c, s, RZ)


_R = 1000


def _dinv_block(dp_ref):
    return lax.rsqrt(dp_ref[0] + dp_ref[1] + 1.0)[:, 0:1]


def _tc_prep(dp, x):
    def body(dp_ref, x_ref, o_ref):
        xsd = x_ref[...] * _dinv_block(dp_ref)
        o_ref[0] = xsd[:, :HD]
        o_ref[1] = xsd[:, HD:]

    return pl.pallas_call(
        body,
        grid=(N // _R,),
        in_specs=[
            pl.BlockSpec((2, _R, DEGW), lambda i: (0, i, 0)),
            pl.BlockSpec((_R, D), lambda i: (i, 0)),
        ],
        out_specs=pl.BlockSpec((2, _R, HD), lambda i: (0, i, 0)),
        out_shape=jax.ShapeDtypeStruct((2, NP, HD), jnp.float32),
    )(dp, x)


def _tc_mm1(dp, ag, x, Wc, bc):
    def body(dp_ref, ag_ref, x_ref, w_ref, b_ref, h1_ref, ys_ref):
        dinv = _dinv_block(dp_ref)
        xsd = x_ref[...] * dinv
        agf = jnp.concatenate([ag_ref[0], ag_ref[1]], axis=1)
        p1 = (agf + xsd) * dinv
        H3 = jnp.dot(p1, w_ref[...], preferred_element_type=jnp.float32)
        H3 = jnp.maximum(H3 + b_ref[...], 0.0)
        h1_ref[...] = H3[:, D:2 * D]
        ys_ref[0] = H3[:, 0:HD] * dinv
        ys_ref[1] = H3[:, HD:D] * dinv
        ys_ref[2] = H3[:, 2 * D:2 * D + HD] * dinv
        ys_ref[3] = H3[:, 2 * D + HD:3 * D] * dinv

    return pl.pallas_call(
        body,
        grid=(N // _R,),
        in_specs=[
            pl.BlockSpec((2, _R, DEGW), lambda i: (0, i, 0)),
            pl.BlockSpec((2, _R, HD), lambda i: (0, i, 0)),
            pl.BlockSpec((_R, D), lambda i: (i, 0)),
            pl.BlockSpec((D, 3 * D), lambda i: (0, 0)),
            pl.BlockSpec((1, 3 * D), lambda i: (0, 0)),
        ],
        out_specs=[
            pl.BlockSpec((_R, D), lambda i: (i, 0)),
            pl.BlockSpec((4, _R, HD), lambda i: (0, i, 0)),
        ],
        out_shape=[
            jax.ShapeDtypeStruct((N, D), jnp.float32),
            jax.ShapeDtypeStruct((4, NP, HD), jnp.float32),
        ],
    )(dp, ag, x, Wc, bc)


def _tc_final(dp, ag2a, ag2b, ys4, h1, Ws2, bs2, Wm22, bm22, wa, ba,
              Wf1, Wf2, bf):
    def body(dp_ref, aga_ref, agb_ref, ys_ref, h1_ref, ws2_ref, bs2_ref,
             wm22_ref, bm22_ref, wa_ref, ba_ref, wf1_ref, wf2_ref, bf_ref,
             o_ref):
        dinv = _dinv_block(dp_ref)
        p2s = jnp.concatenate([aga_ref[0] + ys_ref[0],
                               aga_ref[1] + ys_ref[1]], axis=1) * dinv
        hs = jnp.dot(p2s, ws2_ref[...], preferred_element_type=jnp.float32)
        hs = jnp.maximum(hs + bs2_ref[...], 0.0)
        p2m = jnp.concatenate([agb_ref[0] + ys_ref[2],
                               agb_ref[1] + ys_ref[3]], axis=1) * dinv
        h2 = jnp.dot(p2m, wm22_ref[...], preferred_element_type=jnp.float32)
        h2 = jnp.maximum(h2 + bm22_ref[...], 0.0)
        h1v = h1_ref[...]
        s1 = jnp.sum(h1v * wa_ref[...], axis=1, keepdims=True) + ba_ref[0, 0]
        s2 = jnp.sum(h2 * wa_ref[...], axis=1, keepdims=True) + ba_ref[0, 0]
        m = jnp.maximum(s1, s2)
        e1 = jnp.exp(s1 - m)
        e2 = jnp.exp(s2 - m)
        h_meta = (e1 * h1v + e2 * h2) / (e1 + e2)
        out = jnp.dot(hs, wf1_ref[...], preferred_element_type=jnp.float32)
        out = out + jnp.dot(h_meta, wf2_ref[...],
                            preferred_element_type=jnp.float32)
        o_ref[...] = out + bf_ref[...]

    full = lambda shape: pl.BlockSpec(shape, lambda i: tuple(0 for _ in shape))
    return pl.pallas_call(
        body,
        grid=(N // _R,),
        in_specs=[
            pl.BlockSpec((2, _R, DEGW), lambda i: (0, i, 0)),
            pl.BlockSpec((2, _R, HD), lambda i: (0, i, 0)),
            pl.BlockSpec((2, _R, HD), lambda i: (0, i, 0)),
            pl.BlockSpec((4, _R, HD), lambda i: (0, i, 0)),
            pl.BlockSpec((_R, D), lambda i: (i, 0)),
            full((D, D)),
            full((1, D)),
            full((D, D)),
            full((1, D)),
            full((1, D)),
            full((1, 1)),
            full((D, D)),
            full((D, D)),
            full((1, D)),
        ],
        out_specs=pl.BlockSpec((_R, D), lambda i: (i, 0)),
        out_shape=jax.ShapeDtypeStruct((N, D), jnp.float32),
    )(dp, ag2a, ag2b, ys4, h1, Ws2, bs2, Wm22, bm22, wa, ba, Wf1, Wf2, bf)


def kernel(x, edge_index, W_s1, b_s1, W_s2, b_s2, W_m1, b_m1, W_m21, b_m21,
           W_m22, b_m22, W_attn, b_attn, W_fc, b_fc):
    src = edge_index[0].astype(jnp.int32)
    dst = edge_index[1].astype(jnp.int32)
    pad = EP - E
    src_p = jnp.concatenate([src, jnp.zeros((pad,), jnp.int32)])
    dst_p = jnp.concatenate([dst, jnp.full((pad,), N, jnp.int32)])
    srcW = src_p.reshape(NS, EPW)
    srcA = jnp.stack([srcW, srcW + NP])
    srcBo = srcA + 2 * NP
    dstS = dst_p.reshape(NS, ITERS, CH)
    dstD = dst_p.reshape(NW, ITD, CHD)

    ones16 = jnp.ones((CHD, DEGW), jnp.float32)
    zer16 = jnp.zeros((RZ, DEGW), jnp.float32)
    zer64 = jnp.zeros((RZ, HD), jnp.float32)

    dp = _sc_degree(dstD, ones16, zer16)
    xs1 = _tc_prep(dp, x)
    ag1 = _sc_agg(xs1.reshape(2 * NP, HD), srcA, dstS, zer64)

    Wc = jnp.concatenate([W_s1, W_m1, W_m21], axis=1)
    bc = jnp.concatenate([b_s1, b_m1, b_m21]).reshape(1, 3 * D)
    h1, ys4 = _tc_mm1(dp, ag1, x, Wc, bc)

    ysf = ys4.reshape(4 * NP, HD)
    ag2a = _sc_agg(ysf, srcA, dstS, zer64)
    ag2b = _sc_agg(ysf, srcBo, dstS, zer64)

    out = _tc_final(
        dp, ag2a, ag2b, ys4, h1,
        W_s2, b_s2.reshape(1, D), W_m22, b_m22.reshape(1, D),
        W_attn.reshape(1, D), b_attn.reshape(1, 1),
        W_fc[:D], W_fc[D:], b_fc.reshape(1, D))
    return out


# speedup vs baseline: 1.6205x; 1.6205x over previous
import functools

import jax
import jax.numpy as jnp
from jax import lax
from jax.experimental import pallas as pl
from jax.experimental.pallas import tpu as pltpu
from jax.experimental.pallas import tpu_sc as plsc

N = 10000
NP = 10240
E = 320000
D = 128
HD = 64
NC = 2
NS = 16
NW = NC * NS
CH = 128
K = 8
EP = NW * 10240
EPW = EP // NS
ITERS = EPW // CH
RPT = NP // NS
RZ = 128
DEGW = 16
CHD = 128
EPWD = EP // NW
ITD = EPWD // CHD

_MESH = dict(core_axis_name="c", subcore_axis_name="s")


def _zero_acc(zer_h, buf_v, acc_sh, s, rz):
    pltpu.sync_copy(zer_h, buf_v)
    for k in range(RPT // rz):
        pltpu.sync_copy(buf_v, acc_sh.at[pl.ds(s * RPT + k * rz, rz)])


def _read_out(out_h, buf_v, acc_sh, c, s, rz):
    for k in range(RPT // rz):
        b = s * RPT + k * rz
        pltpu.sync_copy(acc_sh.at[pl.ds(b, rz)], buf_v)
        pltpu.sync_copy(buf_v, out_h.at[c, pl.ds(b, rz)])


@functools.partial(
    pl.kernel,
    out_type=jax.ShapeDtypeStruct((NC, NP, DEGW), jnp.float32),
    mesh=plsc.VectorSubcoreMesh(**_MESH),
    scratch_types=[
        pltpu.VMEM((ITD, CHD), jnp.int32),
        pltpu.VMEM((CHD, DEGW), jnp.float32),
        pltpu.VMEM((RZ, DEGW), jnp.float32),
        pltpu.VMEM_SHARED((NP, DEGW), jnp.float32),
        pltpu.SemaphoreType.DMA,
    ],
    compiler_params=pltpu.CompilerParams(use_tc_tiling_on_sc=False),
    name="sc_degree",
)
def _sc_degree(dst_h, ones_h, zer_h, out_h, dstb_v, ones_v, buf_v, acc_sh,
               ssem):
    c = lax.axis_index("c")
    s = lax.axis_index("s")
    wid = s * NC + c
    pltpu.sync_copy(ones_h, ones_v)
    pltpu.sync_copy(dst_h.at[wid], dstb_v)
    _zero_acc(zer_h, buf_v, acc_sh, s, RZ)
    plsc.subcore_barrier()

    def step(g, carry):
        descs = [
            pltpu.async_copy(ones_v, acc_sh.at[dstb_v.at[g * K + b]], ssem,
                             add=True)
            for b in range(K)
        ]
        for d in descs:
            d.wait()
        return carry

    lax.fori_loop(0, ITD // K, step, 0)
    plsc.subcore_barrier()
    _read_out(out_h, buf_v, acc_sh, c, s, RZ)


@functools.partial(
    pl.kernel,
    out_type=jax.ShapeDtypeStruct((NC, NP, HD), jnp.bfloat16),
    mesh=plsc.VectorSubcoreMesh(**_MESH),
    scratch_types=[
        pltpu.VMEM((EPW,), jnp.int32),
        pltpu.VMEM((ITERS, CH), jnp.int32),
        [pltpu.VMEM((CH, HD), jnp.bfloat16)] * 2,
        pltpu.VMEM_SHARED((NP, HD), jnp.bfloat16),
        pltpu.SemaphoreType.DMA,
        pltpu.SemaphoreType.DMA,
        pltpu.SemaphoreType.DMA,
        pltpu.SemaphoreType.DMA,
    ],
    compiler_params=pltpu.CompilerParams(use_tc_tiling_on_sc=False),
    name="sc_agg",
)
def _sc_agg(tab_h, src_h, dst_h, zer_h, out_h,
            srcb_v, dstb_v, rows, acc_sh, gsemA, gsemB, ssemA, ssemB):
    c = lax.axis_index("c")
    s = lax.axis_index("s")
    pltpu.sync_copy(src_h.at[c, s], srcb_v)
    pltpu.sync_copy(dst_h.at[s], dstb_v)
    _zero_acc(zer_h, rows[0], acc_sh, s, RZ)
    plsc.subcore_barrier()

    def step(i, carry):
        k0 = 2 * i
        k1 = 2 * i + 1
        gA = pltpu.async_copy(
            tab_h.at[srcb_v.at[pl.ds(k0 * CH, CH)]], rows[0], gsemA)
        gB = pltpu.async_copy(
            tab_h.at[srcb_v.at[pl.ds(k1 * CH, CH)]], rows[1], gsemB)
        gA.wait()
        sA = pltpu.async_copy(rows[0], acc_sh.at[dstb_v.at[k0]], ssemA,
                              add=True)
        gB.wait()
        sB = pltpu.async_copy(rows[1], acc_sh.at[dstb_v.at[k1]], ssemB,
                              add=True)
        sA.wait()
        sB.wait()
        return carry

    lax.fori_loop(0, ITERS // 2, step, 0)
    plsc.subcore_barrier()
    _read_out(out_h, rows[0], acc_sh, c, s, RZ)


_R = 1000


def _dinv_block(dp_ref):
    return lax.rsqrt(dp_ref[0] + dp_ref[1] + 1.0)[:, 0:1]


def _tc_prep(dp, x):
    def body(dp_ref, x_ref, o_ref):
        xsd = (x_ref[...] * _dinv_block(dp_ref)).astype(jnp.bfloat16)
        o_ref[0] = xsd[:, :HD]
        o_ref[1] = xsd[:, HD:]

    return pl.pallas_call(
        body,
        grid=(N // _R,),
        in_specs=[
            pl.BlockSpec((2, _R, DEGW), lambda i: (0, i, 0)),
            pl.BlockSpec((_R, D), lambda i: (i, 0)),
        ],
        out_specs=pl.BlockSpec((2, _R, HD), lambda i: (0, i, 0)),
        out_shape=jax.ShapeDtypeStruct((2, NP, HD), jnp.bfloat16),
    )(dp, x)


def _tc_mm1(dp, ag, x, Wc, bc):
    def body(dp_ref, ag_ref, x_ref, w_ref, b_ref, h1_ref, ys_ref):
        dinv = _dinv_block(dp_ref)
        xsd = x_ref[...] * dinv
        agf = jnp.concatenate([ag_ref[0], ag_ref[1]],
                              axis=1).astype(jnp.float32)
        p1 = (agf + xsd) * dinv
        H3 = jnp.dot(p1, w_ref[...], preferred_element_type=jnp.float32)
        H3 = jnp.maximum(H3 + b_ref[...], 0.0)
        h1_ref[...] = H3[:, D:2 * D]
        ysd = (H3 * dinv).astype(jnp.bfloat16)
        ys_ref[0] = ysd[:, 0:HD]
        ys_ref[1] = ysd[:, HD:D]
        ys_ref[2] = ysd[:, 2 * D:2 * D + HD]
        ys_ref[3] = ysd[:, 2 * D + HD:3 * D]

    return pl.pallas_call(
        body,
        grid=(N // _R,),
        in_specs=[
            pl.BlockSpec((2, _R, DEGW), lambda i: (0, i, 0)),
            pl.BlockSpec((2, _R, HD), lambda i: (0, i, 0)),
            pl.BlockSpec((_R, D), lambda i: (i, 0)),
            pl.BlockSpec((D, 3 * D), lambda i: (0, 0)),
            pl.BlockSpec((1, 3 * D), lambda i: (0, 0)),
        ],
        out_specs=[
            pl.BlockSpec((_R, D), lambda i: (i, 0)),
            pl.BlockSpec((4, _R, HD), lambda i: (0, i, 0)),
        ],
        out_shape=[
            jax.ShapeDtypeStruct((N, D), jnp.float32),
            jax.ShapeDtypeStruct((4, NP, HD), jnp.bfloat16),
        ],
    )(dp, ag, x, Wc, bc)


def _tc_final(dp, ag2a, ag2b, ys4, h1, Ws2, bs2, Wm22, bm22, wa, ba,
              Wf1, Wf2, bf):
    def body(dp_ref, aga_ref, agb_ref, ys_ref, h1_ref, ws2_ref, bs2_ref,
             wm22_ref, bm22_ref, wa_ref, ba_ref, wf1_ref, wf2_ref, bf_ref,
             o_ref):
        dinv = _dinv_block(dp_ref)
        aga = jnp.concatenate([aga_ref[0], aga_ref[1]],
                              axis=1).astype(jnp.float32)
        agb = jnp.concatenate([agb_ref[0], agb_ref[1]],
                              axis=1).astype(jnp.float32)
        ys01 = jnp.concatenate([ys_ref[0], ys_ref[1]],
                               axis=1).astype(jnp.float32)
        ys23 = jnp.concatenate([ys_ref[2], ys_ref[3]],
                               axis=1).astype(jnp.float32)
        p2s = (aga + ys01) * dinv
        hs = jnp.dot(p2s, ws2_ref[...], preferred_element_type=jnp.float32)
        hs = jnp.maximum(hs + bs2_ref[...], 0.0)
        p2m = (agb + ys23) * dinv
        h2 = jnp.dot(p2m, wm22_ref[...], preferred_element_type=jnp.float32)
        h2 = jnp.maximum(h2 + bm22_ref[...], 0.0)
        h1v = h1_ref[...]
        s1 = jnp.sum(h1v * wa_ref[...], axis=1, keepdims=True) + ba_ref[0, 0]
        s2 = jnp.sum(h2 * wa_ref[...], axis=1, keepdims=True) + ba_ref[0, 0]
        m = jnp.maximum(s1, s2)
        e1 = jnp.exp(s1 - m)
        e2 = jnp.exp(s2 - m)
        h_meta = (e1 * h1v + e2 * h2) / (e1 + e2)
        out = jnp.dot(hs, wf1_ref[...], preferred_element_type=jnp.float32)
        out = out + jnp.dot(h_meta, wf2_ref[...],
                            preferred_element_type=jnp.float32)
        o_ref[...] = out + bf_ref[...]

    full = lambda shape: pl.BlockSpec(shape, lambda i: tuple(0 for _ in shape))
    return pl.pallas_call(
        body,
        grid=(N // _R,),
        in_specs=[
            pl.BlockSpec((2, _R, DEGW), lambda i: (0, i, 0)),
            pl.BlockSpec((2, _R, HD), lambda i: (0, i, 0)),
            pl.BlockSpec((2, _R, HD), lambda i: (0, i, 0)),
            pl.BlockSpec((4, _R, HD), lambda i: (0, i, 0)),
            pl.BlockSpec((_R, D), lambda i: (i, 0)),
            full((D, D)),
            full((1, D)),
            full((D, D)),
            full((1, D)),
            full((1, D)),
            full((1, 1)),
            full((D, D)),
            full((D, D)),
            full((1, D)),
        ],
        out_specs=pl.BlockSpec((_R, D), lambda i: (i, 0)),
        out_shape=jax.ShapeDtypeStruct((N, D), jnp.float32),
    )(dp, ag2a, ag2b, ys4, h1, Ws2, bs2, Wm22, bm22, wa, ba, Wf1, Wf2, bf)


def kernel(x, edge_index, W_s1, b_s1, W_s2, b_s2, W_m1, b_m1, W_m21, b_m21,
           W_m22, b_m22, W_attn, b_attn, W_fc, b_fc):
    src = edge_index[0].astype(jnp.int32)
    dst = edge_index[1].astype(jnp.int32)
    pad = EP - E
    src_p = jnp.concatenate([src, jnp.zeros((pad,), jnp.int32)])
    dst_p = jnp.concatenate([dst, jnp.full((pad,), N, jnp.int32)])
    srcW = src_p.reshape(NS, EPW)
    srcA = jnp.stack([srcW, srcW + NP])
    srcBo = srcA + 2 * NP
    dstS = dst_p.reshape(NS, ITERS, CH)
    dstD = dst_p.reshape(NW, ITD, CHD)

    ones16 = jnp.ones((CHD, DEGW), jnp.float32)
    zer16 = jnp.zeros((RZ, DEGW), jnp.float32)
    zer64 = jnp.zeros((RZ, HD), jnp.bfloat16)

    dp = _sc_degree(dstD, ones16, zer16)
    xs1 = _tc_prep(dp, x)
    ag1 = _sc_agg(xs1.reshape(2 * NP, HD), srcA, dstS, zer64)

    Wc = jnp.concatenate([W_s1, W_m1, W_m21], axis=1)
    bc = jnp.concatenate([b_s1, b_m1, b_m21]).reshape(1, 3 * D)
    h1, ys4 = _tc_mm1(dp, ag1, x, Wc, bc)

    ysf = ys4.reshape(4 * NP, HD)
    ag2a = _sc_agg(ysf, srcA, dstS, zer64)
    ag2b = _sc_agg(ysf, srcBo, dstS, zer64)

    out = _tc_final(
        dp, ag2a, ag2b, ys4, h1,
        W_s2, b_s2.reshape(1, D), W_m22, b_m22.reshape(1, D),
        W_attn.reshape(1, D), b_attn.reshape(1, 1),
        W_fc[:D], W_fc[D:], b_fc.reshape(1, D))
    return out
